# baseline (speedup 1.0000x reference)
import functools

import jax
import jax.numpy as jnp
from jax.experimental import pallas as pl
from jax.experimental.pallas import tpu as pltpu

N_MAX = 256
N_HEADS = 4
D_HEAD = 128
ALPHA = 0.08838834764831843
ROW_F = N_HEADS * 3 * D_HEAD
OUT_F = N_HEADS * D_HEAD
HALF = N_MAX // 2
PANEL = N_MAX + HALF


def _silu_mask(s, m):
    a = s * jax.nn.sigmoid(s)
    return jnp.where(m, a, 0.0).astype(jnp.bfloat16)


def _block_kernel(rec_ref, rep_ref, x_hbm, tail_hbm, o_ref, buf, sem,
                  *, nb, lp_rows):
    b = pl.program_id(0)
    base = b * N_MAX

    def start_in(blk, sl):
        @pl.when(blk < nb - 1)
        def _():
            pltpu.make_async_copy(x_hbm.at[pl.ds(blk * N_MAX, N_MAX), :],
                                  buf.at[sl], sem.at[sl]).start()

        @pl.when(blk == nb - 1)
        def _():
            pltpu.make_async_copy(tail_hbm, buf.at[sl], sem.at[sl]).start()

    @pl.when(b == 0)
    def _():
        start_in(0, 0)

    @pl.when(b + 1 < nb)
    def _():
        start_in(b + 1, jax.lax.rem(b + 1, 3))

    cur = jax.lax.rem(b, 3)
    pltpu.make_async_copy(buf.at[cur], buf.at[cur], sem.at[cur]).wait()

    xc = buf[cur]
    xp = buf[jax.lax.rem(b + 2, 3)]

    upper = jnp.concatenate([rep_ref[0], rec_ref[0]], axis=1) - base

    gi = jax.lax.broadcasted_iota(jnp.int32, (HALF, PANEL), 0)
    ci = jax.lax.broadcasted_iota(jnp.int32, (HALF, PANEL), 1)
    causal = gi + N_MAX >= ci
    mask_t = causal & (gi < upper[:, :PANEL])
    mask_b = causal & (gi + HALF < upper[:, HALF:])

    vrow = jax.lax.broadcasted_iota(jnp.int32, (2 * N_MAX, D_HEAD), 0)
    vg = vrow + (base - N_MAX)
    vok = (vg >= 0) & (vg < lp_rows)

    alpha = jnp.bfloat16(ALPHA)
    vscale = jnp.bfloat16(1.0 / N_MAX)

    for h in range(N_HEADS):
        o = h * 3 * D_HEAD
        q = xc[:, o:o + D_HEAD] * alpha
        k = jnp.concatenate(
            [xp[:, o + D_HEAD:o + 2 * D_HEAD], xc[:, o + D_HEAD:o + 2 * D_HEAD]],
            axis=0)
        v = jnp.concatenate(
            [xp[:, o + 2 * D_HEAD:o + 3 * D_HEAD], xc[:, o + 2 * D_HEAD:o + 3 * D_HEAD]],
            axis=0)
        v = jnp.where(vok, v * vscale, jnp.bfloat16(0))

        s_t = jax.lax.dot_general(q[:HALF], k[:PANEL],
                                  (((1,), (1,)), ((), ())),
                                  preferred_element_type=jnp.float32)
        o_t = jax.lax.dot_general(_silu_mask(s_t, mask_t), v[:PANEL],
                                  (((1,), (0,)), ((), ())),
                                  preferred_element_type=jnp.float32)

        s_b = jax.lax.dot_general(q[HALF:], k[HALF:],
                                  (((1,), (1,)), ((), ())),
                                  preferred_element_type=jnp.float32)
        o_b = jax.lax.dot_general(_silu_mask(s_b, mask_b), v[HALF:],
                                  (((1,), (0,)), ((), ())),
                                  preferred_element_type=jnp.float32)

        hs = h * D_HEAD
        o_ref[:HALF, hs:hs + D_HEAD] = o_t.astype(jnp.bfloat16)
        o_ref[HALF:, hs:hs + D_HEAD] = o_b.astype(jnp.bfloat16)


@jax.jit
def kernel(qkv, seq_offsets, timestamps, tw, pw):
    L = qkv.shape[0]
    nb = (L + N_MAX - 1) // N_MAX
    x = qkv.reshape(L, ROW_F)

    trows = L - (nb - 1) * N_MAX
    tail = jnp.zeros((N_MAX, ROW_F), jnp.bfloat16).at[:trows].set(
        x[(nb - 1) * N_MAX:])

    offs = seq_offsets.astype(jnp.int32)
    lengths = offs[1:] - offs[:-1]
    row_end = jnp.repeat(offs[1:], lengths, total_repeat_length=L)
    row_end = jnp.pad(row_end, (0, nb * N_MAX - L))
    re3 = row_end.reshape(nb, 1, N_MAX)
    rp3 = jnp.pad(re3[:-1], ((1, 0), (0, 0), (0, 0)))

    out = pl.pallas_call(
        functools.partial(_block_kernel, nb=nb, lp_rows=L),
        grid=(nb,),
        in_specs=[
            pl.BlockSpec((1, 1, N_MAX), lambda b: (b, 0, 0)),
            pl.BlockSpec((1, 1, N_MAX), lambda b: (b, 0, 0)),
            pl.BlockSpec(memory_space=pl.ANY),
            pl.BlockSpec(memory_space=pl.ANY),
        ],
        out_specs=pl.BlockSpec((N_MAX, OUT_F), lambda b: (b, 0)),
        out_shape=jax.ShapeDtypeStruct((L, OUT_F), jnp.bfloat16),
        scratch_shapes=[
            pltpu.VMEM((3, N_MAX, ROW_F), jnp.bfloat16),
            pltpu.SemaphoreType.DMA((3,)),
        ],
        compiler_params=pltpu.CompilerParams(
            dimension_semantics=("arbitrary",),
        ),
    )(re3, rp3, x, tail)
    return out.reshape(L, N_HEADS, D_HEAD)
